# Initial kernel scaffold; baseline (speedup 1.0000x reference)
import jax
import jax.numpy as jnp
from jax.experimental import pallas as pl


def kernel(input_ids, word_table, pos_table, sent_table, gamma, beta):
    raise NotImplementedError("write your pallas kernel here")


# speedup vs baseline: 1.5263x; 1.5263x over previous
import functools

import jax
import jax.numpy as jnp
from jax import lax
from jax.experimental import pallas as pl
from jax.experimental.pallas import tpu as pltpu
from jax.experimental.pallas import tpu_sc as plsc

B, NS, NT, HID = 16, 4, 512, 128
ROWS = B * NS * NT
NC, NSUB = 2, 16
NW = NC * NSUB
RPW = ROWS // NW
CHUNK = 128
NCHUNK = RPW // CHUNK
SENT_PER_W = RPW // NT
EPS = 1e-12


def _sc_gather(ids_hbm, sent_ids_hbm, word_hbm, sent_hbm, out_hbm,
               idx_v, sidx_v, buf, sbuf, sem, ssem):
    wid = lax.axis_index("s") * NC + lax.axis_index("c")
    base = wid * RPW
    pltpu.sync_copy(ids_hbm.at[wid], idx_v)
    pltpu.sync_copy(sent_ids_hbm.at[wid], sidx_v)
    for j in range(NCHUNK):
        pltpu.async_copy(word_hbm.at[idx_v.at[j]], buf, sem).wait()
        pltpu.sync_copy(buf, out_hbm.at[pl.ds(base + j * CHUNK, CHUNK)])
    pltpu.async_copy(sent_hbm.at[sidx_v], sbuf, ssem).wait()
    for s in range(SENT_PER_W):
        pltpu.sync_copy(sbuf.at[pl.ds(4 * s, 1)],
                        out_hbm.at[pl.ds(base + s * NT, 1)])


@functools.lru_cache(maxsize=None)
def _sc_gather_call():
    return pl.kernel(
        _sc_gather,
        out_type=jax.ShapeDtypeStruct((ROWS, HID), jnp.float32),
        mesh=plsc.VectorSubcoreMesh(
            core_axis_name="c", subcore_axis_name="s",
            num_cores=NC, num_subcores=NSUB),
        scratch_types=[
            pltpu.VMEM((NCHUNK, CHUNK), jnp.int32),
            pltpu.VMEM((8,), jnp.int32),
            pltpu.VMEM((CHUNK, HID), jnp.float32),
            pltpu.VMEM((8, HID), jnp.float32),
            pltpu.SemaphoreType.DMA,
            pltpu.SemaphoreType.DMA,
        ],
    )


def _tc_ln(x_ref, pos_ref, g_ref, b_ref, o_ref):
    x = x_ref[...]
    pos = pos_ref[...]
    t = lax.broadcasted_iota(jnp.int32, (NT, 1), 0)
    pos = jnp.where(t > 0, pos, 0.0)
    x = x + pos[None]
    u = jnp.mean(x, axis=-1, keepdims=True)
    d = x - u
    s = jnp.mean(d * d, axis=-1, keepdims=True)
    xn = d * lax.rsqrt(s + EPS)
    o_ref[...] = xn * g_ref[...] + b_ref[...]


def kernel(input_ids, word_table, pos_table, sent_table, gamma, beta):
    ids_flat = input_ids.reshape(NW, NCHUNK, CHUNK)
    sent_ids = jnp.repeat(input_ids[:, :, 0].reshape(NW, SENT_PER_W), 4, axis=1)
    gathered = _sc_gather_call()(ids_flat, sent_ids, word_table, sent_table)

    nsent = B * NS
    R = 4
    out = pl.pallas_call(
        _tc_ln,
        grid=(nsent // R,),
        in_specs=[
            pl.BlockSpec((R, NT, HID), lambda i: (i, 0, 0)),
            pl.BlockSpec((NT, HID), lambda i: (0, 0)),
            pl.BlockSpec((1, HID), lambda i: (0, 0)),
            pl.BlockSpec((1, HID), lambda i: (0, 0)),
        ],
        out_specs=pl.BlockSpec((R, NT, HID), lambda i: (i, 0, 0)),
        out_shape=jax.ShapeDtypeStruct((nsent, NT, HID), jnp.float32),
    )(gathered.reshape(nsent, NT, HID), pos_table,
      gamma.reshape(1, HID), beta.reshape(1, HID))
    return out.reshape(B, NS, NT, HID)


# speedup vs baseline: 1.8114x; 1.1868x over previous
import functools

import jax
import jax.numpy as jnp
from jax import lax
from jax.experimental import pallas as pl
from jax.experimental.pallas import tpu as pltpu
from jax.experimental.pallas import tpu_sc as plsc

B, NS, NT, HID = 16, 4, 512, 128
ROWS = B * NS * NT
NC, NSUB = 2, 16
NW = NC * NSUB
RPW = ROWS // NW
CHUNK = 128
NCHUNK = RPW // CHUNK
SENT_PER_W = RPW // NT
EPS = 1e-12


NBUF = 4


def _sc_gather(ids_hbm, sent_ids_hbm, word_hbm, sent_hbm, out_hbm,
               idx_v, sidx_v, b0, b1, b2, b3, sbuf,
               g0, g1, g2, g3, w0, w1, w2, w3, ssem):
    bufs = [b0, b1, b2, b3]
    gsem = [g0, g1, g2, g3]
    wsem = [w0, w1, w2, w3]
    wid = lax.axis_index("s") * NC + lax.axis_index("c")
    base = wid * RPW
    pltpu.sync_copy(ids_hbm.at[wid], idx_v)
    pltpu.sync_copy(sent_ids_hbm.at[wid], sidx_v)
    scp = pltpu.async_copy(sent_hbm.at[sidx_v], sbuf, ssem)
    gh = [pltpu.async_copy(word_hbm.at[idx_v.at[j]], bufs[j], gsem[j])
          for j in range(NBUF)]
    scp.wait()
    wh = [None] * NBUF
    for j in range(NCHUNK):
        b = j % NBUF
        gh[b].wait()
        wh[b] = pltpu.async_copy(
            bufs[b], out_hbm.at[pl.ds(base + j * CHUNK, CHUNK)], wsem[b])
        if j + NBUF < NCHUNK:
            wh[b].wait()
            gh[b] = pltpu.async_copy(
                word_hbm.at[idx_v.at[j + NBUF]], bufs[b], gsem[b])
    for b in range(NBUF):
        wh[b].wait()
    for s in range(SENT_PER_W):
        pltpu.sync_copy(sbuf.at[pl.ds(4 * s, 1)],
                        out_hbm.at[pl.ds(base + s * NT, 1)])


@functools.lru_cache(maxsize=None)
def _sc_gather_call():
    return pl.kernel(
        _sc_gather,
        out_type=jax.ShapeDtypeStruct((ROWS, HID), jnp.float32),
        mesh=plsc.VectorSubcoreMesh(
            core_axis_name="c", subcore_axis_name="s",
            num_cores=NC, num_subcores=NSUB),
        scratch_types=(
            [pltpu.VMEM((NCHUNK, CHUNK), jnp.int32),
             pltpu.VMEM((8,), jnp.int32)]
            + [pltpu.VMEM((CHUNK, HID), jnp.float32)] * NBUF
            + [pltpu.VMEM((8, HID), jnp.float32)]
            + [pltpu.SemaphoreType.DMA] * (2 * NBUF + 1)
        ),
    )


def _tc_ln(x_ref, pos_ref, g_ref, b_ref, o_ref):
    x = x_ref[...]
    pos = pos_ref[...]
    t = lax.broadcasted_iota(jnp.int32, (NT, 1), 0)
    pos = jnp.where(t > 0, pos, 0.0)
    x = x + pos[None]
    u = jnp.mean(x, axis=-1, keepdims=True)
    d = x - u
    s = jnp.mean(d * d, axis=-1, keepdims=True)
    xn = d * lax.rsqrt(s + EPS)
    o_ref[...] = xn * g_ref[...] + b_ref[...]


def kernel(input_ids, word_table, pos_table, sent_table, gamma, beta):
    ids_flat = input_ids.reshape(NW, NCHUNK, CHUNK)
    sent_ids = jnp.repeat(input_ids[:, :, 0].reshape(NW, SENT_PER_W), 4, axis=1)
    gathered = _sc_gather_call()(ids_flat, sent_ids, word_table, sent_table)

    nsent = B * NS
    R = 8
    out = pl.pallas_call(
        _tc_ln,
        grid=(nsent // R,),
        in_specs=[
            pl.BlockSpec((R, NT, HID), lambda i: (i, 0, 0)),
            pl.BlockSpec((NT, HID), lambda i: (0, 0)),
            pl.BlockSpec((1, HID), lambda i: (0, 0)),
            pl.BlockSpec((1, HID), lambda i: (0, 0)),
        ],
        out_specs=pl.BlockSpec((R, NT, HID), lambda i: (i, 0, 0)),
        out_shape=jax.ShapeDtypeStruct((nsent, NT, HID), jnp.float32),
    )(gathered.reshape(nsent, NT, HID), pos_table,
      gamma.reshape(1, HID), beta.reshape(1, HID))
    return out.reshape(B, NS, NT, HID)
